# baseline (device time: 82526 ns/iter reference)
import jax
import jax.numpy as jnp
from jax import lax
from jax.experimental import pallas as pl
from jax.experimental.pallas import tpu as pltpu

N_DEV = 8
SQ = 1024
SKV = 1024
H_LOC = 8
DH = 128
D_LOC = H_LOC * DH
BLK = 64
SCALE = 0.08838834764831843

ORDERS = [[1, 3, 4], [3, 4, 1], [4, 1, 3]]
COLS = [(0, 384), (384, 384), (768, 256)]


def kernel(x, Wq, K_ext, V_ext, Wo):
    my = lax.axis_index("i")
    Wq_loc = lax.dynamic_slice(Wq, (0, my * D_LOC), (Wq.shape[0], D_LOC))
    Wo_loc = lax.dynamic_slice(Wo, (my * D_LOC, 0), (D_LOC, Wo.shape[1]))

    def body(x_ref, wq_ref, k_ref, v_ref, wo_ref, out_ref,
             comm_ref, ctx_ref, rs_send, rs_recv, ag_send, ag_recv):
        my_pos = lax.axis_index("i")
        b0 = my_pos % 2
        b1 = (my_pos // 2) % 2
        b2 = my_pos // 4
        dual = {1: b0 ^ b1, 3: b1, 4: b2}

        barrier_sem = pltpu.get_barrier_semaphore()
        for m in (1, 3, 4):
            pl.semaphore_signal(barrier_sem, inc=1,
                                device_id=(my_pos ^ m,),
                                device_id_type=pl.DeviceIdType.MESH)
        pl.semaphore_wait(barrier_sem, 3)

        bf16 = jnp.bfloat16
        xm = x_ref[0].astype(bf16)
        Q = jnp.dot(xm, wq_ref[...].astype(bf16),
                    preferred_element_type=jnp.float32).astype(bf16)

        HR = SQ // 2
        rbT = lax.broadcasted_iota(jnp.int32, (HR, HR), 0) // BLK
        cbT = lax.broadcasted_iota(jnp.int32, (HR, HR), 1) // BLK
        maskT = cbT <= rbT
        rbB = HR // BLK + lax.broadcasted_iota(jnp.int32, (HR, SKV), 0) // BLK
        cbB = lax.broadcasted_iota(jnp.int32, (HR, SKV), 1) // BLK
        maskB = cbB <= rbB

        for h in range(H_LOC):
            k = k_ref[0, :, h, :].astype(bf16)
            v = v_ref[0, :, h, :].astype(bf16)
            for mask, r0, kl in ((maskT, 0, HR), (maskB, HR, SKV)):
                q = Q[r0:r0 + HR, h * DH:(h + 1) * DH]
                s = lax.dot_general(q, k[:kl, :], (((1,), (1,)), ((), ())),
                                    preferred_element_type=jnp.float32) * SCALE
                s = jnp.where(mask, s, -1e9)
                mx = jnp.max(s, axis=-1, keepdims=True)
                w = jnp.exp(s - mx)
                w = (w / jnp.sum(w, axis=-1, keepdims=True)).astype(bf16)
                ctx_ref[r0:r0 + HR, h * DH:(h + 1) * DH] = jnp.dot(
                    w, v[:kl, :],
                    preferred_element_type=jnp.float32).astype(bf16)

        def vbits(p):
            return [dual[ORDERS[p][j]] for j in range(3)]

        for p in range(3):
            vb = vbits(p)
            c0, cw = COLS[p]
            s_start = (1 - vb[0]) * HR
            out_ref[0, pl.ds(s_start, HR), pl.ds(c0, cw)] = jnp.dot(
                ctx_ref[pl.ds(s_start, HR), :], wo_ref[:, c0:c0 + cw].astype(bf16),
                preferred_element_type=jnp.float32)

        for lvl in range(3):
            half = 512 >> lvl
            rdmas = []
            for p in range(3):
                vb = vbits(p)
                partner = my_pos ^ ORDERS[p][lvl]
                base = sum((vb[j] * (512 >> j) for j in range(lvl)), 0)
                s_start = base + (1 - vb[lvl]) * half
                c0, cw = COLS[p]
                rdma = pltpu.make_async_remote_copy(
                    src_ref=out_ref.at[0, pl.ds(s_start, half), pl.ds(c0, cw)],
                    dst_ref=comm_ref.at[lvl, pl.ds(0, half), pl.ds(c0, cw)],
                    send_sem=rs_send.at[p, lvl],
                    recv_sem=rs_recv.at[p, lvl],
                    device_id=(partner,),
                    device_id_type=pl.DeviceIdType.MESH,
                )
                rdma.start()
                rdmas.append(rdma)
            if lvl == 0:
                for p in range(3):
                    vb = vbits(p)
                    c0, cw = COLS[p]
                    k_start = vb[0] * HR
                    out_ref[0, pl.ds(k_start, HR), pl.ds(c0, cw)] = jnp.dot(
                        ctx_ref[pl.ds(k_start, HR), :],
                        wo_ref[:, c0:c0 + cw].astype(bf16),
                        preferred_element_type=jnp.float32)
            for p in range(3):
                rdmas[p].wait()
                vb = vbits(p)
                base = sum((vb[j] * (512 >> j) for j in range(lvl)), 0)
                k_start = base + vb[lvl] * half
                c0, cw = COLS[p]
                out_ref[0, pl.ds(k_start, half), pl.ds(c0, cw)] = (
                    out_ref[0, pl.ds(k_start, half), pl.ds(c0, cw)]
                    + comm_ref[lvl, pl.ds(0, half), pl.ds(c0, cw)]
                )

        for lvl in range(3):
            g = 128 << lvl
            rs_lvl = 2 - lvl
            rdmas = []
            for p in range(3):
                vb = vbits(p)
                partner = my_pos ^ ORDERS[p][rs_lvl]
                start = sum((vb[j] * (512 >> j) for j in range(rs_lvl + 1)), 0)
                c0, cw = COLS[p]
                rdma = pltpu.make_async_remote_copy(
                    src_ref=out_ref.at[0, pl.ds(start, g), pl.ds(c0, cw)],
                    dst_ref=out_ref.at[0, pl.ds(start, g), pl.ds(c0, cw)],
                    send_sem=ag_send.at[p, lvl],
                    recv_sem=ag_recv.at[p, lvl],
                    device_id=(partner,),
                    device_id_type=pl.DeviceIdType.MESH,
                )
                rdma.start()
                rdmas.append(rdma)
            for p in range(3):
                rdmas[p].wait()

    return pl.pallas_call(
        body,
        out_shape=jax.ShapeDtypeStruct((1, SQ, Wo.shape[1]), jnp.float32),
        in_specs=[pl.BlockSpec(memory_space=pltpu.VMEM)] * 5,
        out_specs=pl.BlockSpec(memory_space=pltpu.VMEM),
        scratch_shapes=[
            pltpu.VMEM((3, 512, 1024), jnp.float32),
            pltpu.VMEM((SQ, 1024), jnp.bfloat16),
            pltpu.SemaphoreType.DMA((3, 3)),
            pltpu.SemaphoreType.DMA((3, 3)),
            pltpu.SemaphoreType.DMA((3, 3)),
            pltpu.SemaphoreType.DMA((3, 3)),
        ],
        compiler_params=pltpu.CompilerParams(collective_id=0),
    )(x, Wq_loc, K_ext, V_ext, Wo_loc)


# device time: 73806 ns/iter; 1.1181x vs baseline; 1.1181x over previous
import jax
import jax.numpy as jnp
from jax import lax
from jax.experimental import pallas as pl
from jax.experimental.pallas import tpu as pltpu

N_DEV = 8
SQ = 1024
SKV = 1024
H_LOC = 8
DH = 128
D_LOC = H_LOC * DH
BLK = 64
SCALE = 0.08838834764831843

ORDERS = [[1, 3, 4], [3, 4, 1], [4, 1, 3]]
COLS = [(0, 384), (384, 384), (768, 256)]


def kernel(x, Wq, K_ext, V_ext, Wo):
    my = lax.axis_index("i")
    Wq_loc = lax.dynamic_slice(Wq, (0, my * D_LOC), (Wq.shape[0], D_LOC))
    Wo_loc = lax.dynamic_slice(Wo, (my * D_LOC, 0), (D_LOC, Wo.shape[1]))

    def body(x_ref, wq_ref, k_ref, v_ref, wo_ref, out_ref,
             comm_ref, ctx_ref, rs_send, rs_recv, ag_send, ag_recv):
        my_pos = lax.axis_index("i")
        b0 = my_pos % 2
        b1 = (my_pos // 2) % 2
        b2 = my_pos // 4
        dual = {1: b0 ^ b1, 3: b1, 4: b2}

        barrier_sem = pltpu.get_barrier_semaphore()
        for m in (1, 3, 4):
            pl.semaphore_signal(barrier_sem, inc=1,
                                device_id=(my_pos ^ m,),
                                device_id_type=pl.DeviceIdType.MESH)
        pl.semaphore_wait(barrier_sem, 3)

        xm = x_ref[0]
        Q = jnp.dot(xm, wq_ref[...], preferred_element_type=jnp.float32) * SCALE

        HR = SQ // 2
        rbT = lax.broadcasted_iota(jnp.int32, (HR, HR), 0) // BLK
        cbT = lax.broadcasted_iota(jnp.int32, (HR, HR), 1) // BLK
        maskT = cbT <= rbT
        rbB = HR // BLK + lax.broadcasted_iota(jnp.int32, (HR, SKV), 0) // BLK
        cbB = lax.broadcasted_iota(jnp.int32, (HR, SKV), 1) // BLK
        maskB = cbB <= rbB

        for h in range(H_LOC):
            k = k_ref[0, :, h, :]
            v = v_ref[0, :, h, :]
            for mask, r0, kl in ((maskT, 0, HR), (maskB, HR, SKV)):
                q = Q[r0:r0 + HR, h * DH:(h + 1) * DH]
                s = lax.dot_general(q, k[:kl, :], (((1,), (1,)), ((), ())),
                                    preferred_element_type=jnp.float32)
                w = jnp.where(mask, jnp.exp(s), 0.0)
                denom = jnp.sum(w, axis=-1, keepdims=True)
                ctxh = jnp.dot(w, v[:kl, :], preferred_element_type=jnp.float32)
                ctx_ref[r0:r0 + HR, h * DH:(h + 1) * DH] = ctxh / denom

        def vbits(p):
            return [dual[ORDERS[p][j]] for j in range(3)]

        for p in range(3):
            vb = vbits(p)
            c0, cw = COLS[p]
            s_start = (1 - vb[0]) * HR
            out_ref[0, pl.ds(s_start, HR), pl.ds(c0, cw)] = jnp.dot(
                ctx_ref[pl.ds(s_start, HR), :], wo_ref[:, c0:c0 + cw],
                preferred_element_type=jnp.float32)

        for lvl in range(3):
            half = 512 >> lvl
            rdmas = []
            for p in range(3):
                vb = vbits(p)
                partner = my_pos ^ ORDERS[p][lvl]
                base = sum((vb[j] * (512 >> j) for j in range(lvl)), 0)
                s_start = base + (1 - vb[lvl]) * half
                c0, cw = COLS[p]
                rdma = pltpu.make_async_remote_copy(
                    src_ref=out_ref.at[0, pl.ds(s_start, half), pl.ds(c0, cw)],
                    dst_ref=comm_ref.at[lvl, pl.ds(0, half), pl.ds(c0, cw)],
                    send_sem=rs_send.at[p, lvl],
                    recv_sem=rs_recv.at[p, lvl],
                    device_id=(partner,),
                    device_id_type=pl.DeviceIdType.MESH,
                )
                rdma.start()
                rdmas.append(rdma)
            if lvl == 0:
                for p in range(3):
                    vb = vbits(p)
                    c0, cw = COLS[p]
                    k_start = vb[0] * HR
                    out_ref[0, pl.ds(k_start, HR), pl.ds(c0, cw)] = jnp.dot(
                        ctx_ref[pl.ds(k_start, HR), :], wo_ref[:, c0:c0 + cw],
                        preferred_element_type=jnp.float32)
            for p in range(3):
                rdmas[p].wait()
                vb = vbits(p)
                base = sum((vb[j] * (512 >> j) for j in range(lvl)), 0)
                k_start = base + vb[lvl] * half
                c0, cw = COLS[p]
                out_ref[0, pl.ds(k_start, half), pl.ds(c0, cw)] = (
                    out_ref[0, pl.ds(k_start, half), pl.ds(c0, cw)]
                    + comm_ref[lvl, pl.ds(0, half), pl.ds(c0, cw)]
                )

        for lvl in range(3):
            g = 128 << lvl
            rs_lvl = 2 - lvl
            rdmas = []
            for p in range(3):
                vb = vbits(p)
                partner = my_pos ^ ORDERS[p][rs_lvl]
                start = sum((vb[j] * (512 >> j) for j in range(rs_lvl + 1)), 0)
                c0, cw = COLS[p]
                rdma = pltpu.make_async_remote_copy(
                    src_ref=out_ref.at[0, pl.ds(start, g), pl.ds(c0, cw)],
                    dst_ref=out_ref.at[0, pl.ds(start, g), pl.ds(c0, cw)],
                    send_sem=ag_send.at[p, lvl],
                    recv_sem=ag_recv.at[p, lvl],
                    device_id=(partner,),
                    device_id_type=pl.DeviceIdType.MESH,
                )
                rdma.start()
                rdmas.append(rdma)
            for p in range(3):
                rdmas[p].wait()

    return pl.pallas_call(
        body,
        out_shape=jax.ShapeDtypeStruct((1, SQ, Wo.shape[1]), jnp.float32),
        in_specs=[pl.BlockSpec(memory_space=pltpu.VMEM)] * 5,
        out_specs=pl.BlockSpec(memory_space=pltpu.VMEM),
        scratch_shapes=[
            pltpu.VMEM((3, 512, 1024), jnp.float32),
            pltpu.VMEM((SQ, 1024), jnp.float32),
            pltpu.SemaphoreType.DMA((3, 3)),
            pltpu.SemaphoreType.DMA((3, 3)),
            pltpu.SemaphoreType.DMA((3, 3)),
            pltpu.SemaphoreType.DMA((3, 3)),
        ],
        compiler_params=pltpu.CompilerParams(collective_id=0),
    )(x, Wq_loc, K_ext, V_ext, Wo_loc)


# device time: 72264 ns/iter; 1.1420x vs baseline; 1.0213x over previous
import jax
import jax.numpy as jnp
from jax import lax
from jax.experimental import pallas as pl
from jax.experimental.pallas import tpu as pltpu

N_DEV = 8
SQ = 1024
SKV = 1024
H_LOC = 8
DH = 128
D_LOC = H_LOC * DH
BLK = 64
SCALE = 0.08838834764831843

ORDERS = [[1, 3, 4], [3, 4, 1], [4, 1, 3]]
COLS = [(0, 384), (384, 384), (768, 256)]


def kernel(x, Wq, K_ext, V_ext, Wo):
    my = lax.axis_index("i")
    Wq_loc = lax.dynamic_slice(Wq, (0, my * D_LOC), (Wq.shape[0], D_LOC))
    Wo_loc = lax.dynamic_slice(Wo, (my * D_LOC, 0), (D_LOC, Wo.shape[1]))

    def body(x_ref, wq_ref, k_ref, v_ref, wo_ref, out_ref,
             comm_ref, ctx_ref, rs_send, rs_recv, ag_send, ag_recv):
        my_pos = lax.axis_index("i")
        b0 = my_pos % 2
        b1 = (my_pos // 2) % 2
        b2 = my_pos // 4
        dual = {1: b0 ^ b1, 3: b1, 4: b2}

        barrier_sem = pltpu.get_barrier_semaphore()
        for m in (1, 3, 4):
            pl.semaphore_signal(barrier_sem, inc=1,
                                device_id=(my_pos ^ m,),
                                device_id_type=pl.DeviceIdType.MESH)
        pl.semaphore_wait(barrier_sem, 3)

        xm = x_ref[0]
        Q = jnp.dot(xm, wq_ref[...], preferred_element_type=jnp.float32) * SCALE

        HR = SQ // 2
        rbT = lax.broadcasted_iota(jnp.int32, (HR, HR), 0) // BLK
        cbT = lax.broadcasted_iota(jnp.int32, (HR, HR), 1) // BLK
        maskT = cbT <= rbT
        rbB = HR // BLK + lax.broadcasted_iota(jnp.int32, (HR, SKV), 0) // BLK
        cbB = lax.broadcasted_iota(jnp.int32, (HR, SKV), 1) // BLK
        maskB = cbB <= rbB

        for h in range(H_LOC):
            k = k_ref[0, :, h, :]
            v = v_ref[0, :, h, :]
            for mask, r0, kl in ((maskT, 0, HR), (maskB, HR, SKV)):
                q = Q[r0:r0 + HR, h * DH:(h + 1) * DH]
                s = lax.dot_general(q, k[:kl, :], (((1,), (1,)), ((), ())),
                                    preferred_element_type=jnp.float32)
                w = jnp.where(mask, jnp.exp(s), 0.0)
                denom = jnp.sum(w, axis=-1, keepdims=True)
                ctxh = jnp.dot(w, v[:kl, :], preferred_element_type=jnp.float32)
                ctx_ref[r0:r0 + HR, h * DH:(h + 1) * DH] = ctxh / denom

        def vbits(p):
            return [dual[ORDERS[p][j]] for j in range(3)]

        def rs_rdma(lvl, p):
            half = 512 >> lvl
            vb = vbits(p)
            base = sum((vb[j] * (512 >> j) for j in range(lvl)), 0)
            s_start = base + (1 - vb[lvl]) * half
            c0, cw = COLS[p]
            return pltpu.make_async_remote_copy(
                src_ref=out_ref.at[0, pl.ds(s_start, half), pl.ds(c0, cw)],
                dst_ref=comm_ref.at[lvl, pl.ds(0, half), pl.ds(c0, cw)],
                send_sem=rs_send.at[p, lvl],
                recv_sem=rs_recv.at[p, lvl],
                device_id=(my_pos ^ ORDERS[p][lvl],),
                device_id_type=pl.DeviceIdType.MESH,
            )

        def rs_add(lvl, p):
            half = 512 >> lvl
            vb = vbits(p)
            base = sum((vb[j] * (512 >> j) for j in range(lvl)), 0)
            k_start = base + vb[lvl] * half
            c0, cw = COLS[p]
            out_ref[0, pl.ds(k_start, half), pl.ds(c0, cw)] = (
                out_ref[0, pl.ds(k_start, half), pl.ds(c0, cw)]
                + comm_ref[lvl, pl.ds(0, half), pl.ds(c0, cw)]
            )

        def ag_rdma(lvl, p):
            g = 128 << lvl
            rs_lvl = 2 - lvl
            vb = vbits(p)
            start = sum((vb[j] * (512 >> j) for j in range(rs_lvl + 1)), 0)
            c0, cw = COLS[p]
            return pltpu.make_async_remote_copy(
                src_ref=out_ref.at[0, pl.ds(start, g), pl.ds(c0, cw)],
                dst_ref=out_ref.at[0, pl.ds(start, g), pl.ds(c0, cw)],
                send_sem=ag_send.at[p, lvl],
                recv_sem=ag_recv.at[p, lvl],
                device_id=(my_pos ^ ORDERS[p][rs_lvl],),
                device_id_type=pl.DeviceIdType.MESH,
            )

        rs_d = {}
        for p in range(3):
            vb = vbits(p)
            c0, cw = COLS[p]
            s_start = (1 - vb[0]) * HR
            out_ref[0, pl.ds(s_start, HR), pl.ds(c0, cw)] = jnp.dot(
                ctx_ref[pl.ds(s_start, HR), :], wo_ref[:, c0:c0 + cw],
                preferred_element_type=jnp.float32)
            rs_d[(0, p)] = rs_rdma(0, p)
            rs_d[(0, p)].start()
        for p in range(3):
            vb = vbits(p)
            c0, cw = COLS[p]
            k_start = vb[0] * HR
            out_ref[0, pl.ds(k_start, HR), pl.ds(c0, cw)] = jnp.dot(
                ctx_ref[pl.ds(k_start, HR), :], wo_ref[:, c0:c0 + cw],
                preferred_element_type=jnp.float32)

        ag_d = {}
        for lvl in range(3):
            for p in range(3):
                rs_d[(lvl, p)].wait()
                rs_add(lvl, p)
                if lvl < 2:
                    rs_d[(lvl + 1, p)] = rs_rdma(lvl + 1, p)
                    rs_d[(lvl + 1, p)].start()
                else:
                    ag_d[(0, p)] = ag_rdma(0, p)
                    ag_d[(0, p)].start()
        for lvl in range(3):
            for p in range(3):
                ag_d[(lvl, p)].wait()
                if lvl < 2:
                    ag_d[(lvl + 1, p)] = ag_rdma(lvl + 1, p)
                    ag_d[(lvl + 1, p)].start()

    return pl.pallas_call(
        body,
        out_shape=jax.ShapeDtypeStruct((1, SQ, Wo.shape[1]), jnp.float32),
        in_specs=[pl.BlockSpec(memory_space=pltpu.VMEM)] * 5,
        out_specs=pl.BlockSpec(memory_space=pltpu.VMEM),
        scratch_shapes=[
            pltpu.VMEM((3, 512, 1024), jnp.float32),
            pltpu.VMEM((SQ, 1024), jnp.float32),
            pltpu.SemaphoreType.DMA((3, 3)),
            pltpu.SemaphoreType.DMA((3, 3)),
            pltpu.SemaphoreType.DMA((3, 3)),
            pltpu.SemaphoreType.DMA((3, 3)),
        ],
        compiler_params=pltpu.CompilerParams(collective_id=0),
    )(x, Wq_loc, K_ext, V_ext, Wo_loc)


# device time: 71389 ns/iter; 1.1560x vs baseline; 1.0123x over previous
import jax
import jax.numpy as jnp
from jax import lax
from jax.experimental import pallas as pl
from jax.experimental.pallas import tpu as pltpu

N_DEV = 8
SQ = 1024
SKV = 1024
H_LOC = 8
DH = 128
D_LOC = H_LOC * DH
BLK = 64
SCALE = 0.08838834764831843

ORDERS = [[1, 3, 4], [3, 4, 1], [4, 1, 3]]
COLS = [(0, 384), (384, 384), (768, 256)]


def kernel(x, Wq, K_ext, V_ext, Wo):
    my = lax.axis_index("i")
    Wq_loc = lax.dynamic_slice(Wq, (0, my * D_LOC), (Wq.shape[0], D_LOC))
    Wo_loc = lax.dynamic_slice(Wo, (my * D_LOC, 0), (D_LOC, Wo.shape[1]))

    def body(x_ref, wq_ref, k_ref, v_ref, wo_ref, out_ref,
             comm_ref, ctx_ref, rs_send, rs_recv, ag_send, ag_recv):
        my_pos = lax.axis_index("i")
        b0 = my_pos % 2
        b1 = (my_pos // 2) % 2
        b2 = my_pos // 4
        dual = {1: b0 ^ b1, 3: b1, 4: b2}

        barrier_sem = pltpu.get_barrier_semaphore()
        for m in (1, 3, 4):
            pl.semaphore_signal(barrier_sem, inc=1,
                                device_id=(my_pos ^ m,),
                                device_id_type=pl.DeviceIdType.MESH)
        pl.semaphore_wait(barrier_sem, 3)

        xm = x_ref[0]
        Q = jnp.dot(xm, wq_ref[...], preferred_element_type=jnp.float32) * SCALE

        HR = SQ // 2
        QR = SQ // 4
        masks = []
        for qi in range(4):
            kl = QR * (qi + 1)
            rbq = qi * (QR // BLK) + lax.broadcasted_iota(
                jnp.int32, (QR, kl), 0) // BLK
            cbq = lax.broadcasted_iota(jnp.int32, (QR, kl), 1) // BLK
            masks.append(cbq <= rbq)

        for h in range(H_LOC):
            k = k_ref[0, :, h, :]
            v = v_ref[0, :, h, :]
            for qi in range(4):
                kl = QR * (qi + 1)
                q = Q[qi * QR:(qi + 1) * QR, h * DH:(h + 1) * DH]
                s = lax.dot_general(q, k[:kl, :], (((1,), (1,)), ((), ())),
                                    preferred_element_type=jnp.float32)
                w = jnp.where(masks[qi], jnp.exp(s), 0.0)
                denom = jnp.sum(w, axis=-1, keepdims=True)
                ctxh = jnp.dot(w, v[:kl, :], preferred_element_type=jnp.float32)
                ctx_ref[qi * QR:(qi + 1) * QR, h * DH:(h + 1) * DH] = ctxh / denom

        def vbits(p):
            return [dual[ORDERS[p][j]] for j in range(3)]

        def rs_rdma(lvl, p):
            half = 512 >> lvl
            vb = vbits(p)
            base = sum((vb[j] * (512 >> j) for j in range(lvl)), 0)
            s_start = base + (1 - vb[lvl]) * half
            c0, cw = COLS[p]
            return pltpu.make_async_remote_copy(
                src_ref=out_ref.at[0, pl.ds(s_start, half), pl.ds(c0, cw)],
                dst_ref=comm_ref.at[lvl, pl.ds(0, half), pl.ds(c0, cw)],
                send_sem=rs_send.at[p, lvl],
                recv_sem=rs_recv.at[p, lvl],
                device_id=(my_pos ^ ORDERS[p][lvl],),
                device_id_type=pl.DeviceIdType.MESH,
            )

        def rs_add(lvl, p):
            half = 512 >> lvl
            vb = vbits(p)
            base = sum((vb[j] * (512 >> j) for j in range(lvl)), 0)
            k_start = base + vb[lvl] * half
            c0, cw = COLS[p]
            out_ref[0, pl.ds(k_start, half), pl.ds(c0, cw)] = (
                out_ref[0, pl.ds(k_start, half), pl.ds(c0, cw)]
                + comm_ref[lvl, pl.ds(0, half), pl.ds(c0, cw)]
            )

        def ag_rdma(lvl, p):
            g = 128 << lvl
            rs_lvl = 2 - lvl
            vb = vbits(p)
            start = sum((vb[j] * (512 >> j) for j in range(rs_lvl + 1)), 0)
            c0, cw = COLS[p]
            return pltpu.make_async_remote_copy(
                src_ref=out_ref.at[0, pl.ds(start, g), pl.ds(c0, cw)],
                dst_ref=out_ref.at[0, pl.ds(start, g), pl.ds(c0, cw)],
                send_sem=ag_send.at[p, lvl],
                recv_sem=ag_recv.at[p, lvl],
                device_id=(my_pos ^ ORDERS[p][rs_lvl],),
                device_id_type=pl.DeviceIdType.MESH,
            )

        rs_d = {}
        for p in range(3):
            vb = vbits(p)
            c0, cw = COLS[p]
            s_start = (1 - vb[0]) * HR
            out_ref[0, pl.ds(s_start, HR), pl.ds(c0, cw)] = jnp.dot(
                ctx_ref[pl.ds(s_start, HR), :], wo_ref[:, c0:c0 + cw],
                preferred_element_type=jnp.float32)
            rs_d[(0, p)] = rs_rdma(0, p)
            rs_d[(0, p)].start()
        for p in range(3):
            vb = vbits(p)
            c0, cw = COLS[p]
            k_start = vb[0] * HR
            out_ref[0, pl.ds(k_start, HR), pl.ds(c0, cw)] = jnp.dot(
                ctx_ref[pl.ds(k_start, HR), :], wo_ref[:, c0:c0 + cw],
                preferred_element_type=jnp.float32)

        ag_d = {}
        for lvl in range(3):
            for p in range(3):
                rs_d[(lvl, p)].wait()
                rs_add(lvl, p)
                if lvl < 2:
                    rs_d[(lvl + 1, p)] = rs_rdma(lvl + 1, p)
                    rs_d[(lvl + 1, p)].start()
                else:
                    ag_d[(0, p)] = ag_rdma(0, p)
                    ag_d[(0, p)].start()
        for lvl in range(3):
            for p in range(3):
                ag_d[(lvl, p)].wait()
                if lvl < 2:
                    ag_d[(lvl + 1, p)] = ag_rdma(lvl + 1, p)
                    ag_d[(lvl + 1, p)].start()

    return pl.pallas_call(
        body,
        out_shape=jax.ShapeDtypeStruct((1, SQ, Wo.shape[1]), jnp.float32),
        in_specs=[pl.BlockSpec(memory_space=pltpu.VMEM)] * 5,
        out_specs=pl.BlockSpec(memory_space=pltpu.VMEM),
        scratch_shapes=[
            pltpu.VMEM((3, 512, 1024), jnp.float32),
            pltpu.VMEM((SQ, 1024), jnp.float32),
            pltpu.SemaphoreType.DMA((3, 3)),
            pltpu.SemaphoreType.DMA((3, 3)),
            pltpu.SemaphoreType.DMA((3, 3)),
            pltpu.SemaphoreType.DMA((3, 3)),
        ],
        compiler_params=pltpu.CompilerParams(collective_id=0),
    )(x, Wq_loc, K_ext, V_ext, Wo_loc)


# device time: 69520 ns/iter; 1.1871x vs baseline; 1.0269x over previous
import jax
import jax.numpy as jnp
from jax import lax
from jax.experimental import pallas as pl
from jax.experimental.pallas import tpu as pltpu

N_DEV = 8
SQ = 1024
SKV = 1024
H_LOC = 8
DH = 128
D_LOC = H_LOC * DH
BLK = 64
SCALE = 0.08838834764831843

ORDERS = [[1, 3, 4], [3, 4, 1], [4, 1, 3]]
COLS = [(0, 384), (384, 384), (768, 256)]


def kernel(x, Wq, K_ext, V_ext, Wo):
    my = lax.axis_index("i")
    Wq_loc = lax.dynamic_slice(Wq, (0, my * D_LOC), (Wq.shape[0], D_LOC))
    Wo_loc = lax.dynamic_slice(Wo, (my * D_LOC, 0), (D_LOC, Wo.shape[1]))

    def body(x_ref, wq_ref, k_hbm, v_hbm, wo_hbm, out_ref,
             comm_ref, ctx_ref, k_ref, v_ref, wo_ref, cp_sems,
             rs_send, rs_recv, ag_send, ag_recv):
        my_pos = lax.axis_index("i")

        cp_k = pltpu.make_async_copy(k_hbm, k_ref, cp_sems.at[0])
        cp_v = pltpu.make_async_copy(v_hbm, v_ref, cp_sems.at[1])
        cp_wo = pltpu.make_async_copy(wo_hbm, wo_ref, cp_sems.at[2])
        cp_k.start()
        cp_v.start()
        cp_wo.start()
        b0 = my_pos % 2
        b1 = (my_pos // 2) % 2
        b2 = my_pos // 4
        dual = {1: b0 ^ b1, 3: b1, 4: b2}

        barrier_sem = pltpu.get_barrier_semaphore()
        for m in (1, 3, 4):
            pl.semaphore_signal(barrier_sem, inc=1,
                                device_id=(my_pos ^ m,),
                                device_id_type=pl.DeviceIdType.MESH)
        pl.semaphore_wait(barrier_sem, 3)

        xm = x_ref[0]
        Q = jnp.dot(xm, wq_ref[...], preferred_element_type=jnp.float32) * SCALE

        HR = SQ // 2
        QR = SQ // 4
        masks = []
        for qi in range(4):
            kl = QR * (qi + 1)
            rbq = qi * (QR // BLK) + lax.broadcasted_iota(
                jnp.int32, (QR, kl), 0) // BLK
            cbq = lax.broadcasted_iota(jnp.int32, (QR, kl), 1) // BLK
            masks.append(cbq <= rbq)

        cp_k.wait()
        cp_v.wait()
        for h in range(H_LOC):
            k = k_ref[0, :, h, :]
            v = v_ref[0, :, h, :]
            for qi in range(4):
                kl = QR * (qi + 1)
                q = Q[qi * QR:(qi + 1) * QR, h * DH:(h + 1) * DH]
                s = lax.dot_general(q, k[:kl, :], (((1,), (1,)), ((), ())),
                                    preferred_element_type=jnp.float32)
                w = jnp.where(masks[qi], jnp.exp(s), 0.0)
                denom = jnp.sum(w, axis=-1, keepdims=True)
                ctxh = jnp.dot(w, v[:kl, :], preferred_element_type=jnp.float32)
                ctx_ref[qi * QR:(qi + 1) * QR, h * DH:(h + 1) * DH] = ctxh / denom

        def vbits(p):
            return [dual[ORDERS[p][j]] for j in range(3)]

        def rs_rdma(lvl, p):
            half = 512 >> lvl
            vb = vbits(p)
            base = sum((vb[j] * (512 >> j) for j in range(lvl)), 0)
            s_start = base + (1 - vb[lvl]) * half
            c0, cw = COLS[p]
            return pltpu.make_async_remote_copy(
                src_ref=out_ref.at[0, pl.ds(s_start, half), pl.ds(c0, cw)],
                dst_ref=comm_ref.at[lvl, pl.ds(0, half), pl.ds(c0, cw)],
                send_sem=rs_send.at[p, lvl],
                recv_sem=rs_recv.at[p, lvl],
                device_id=(my_pos ^ ORDERS[p][lvl],),
                device_id_type=pl.DeviceIdType.MESH,
            )

        def rs_add(lvl, p):
            half = 512 >> lvl
            vb = vbits(p)
            base = sum((vb[j] * (512 >> j) for j in range(lvl)), 0)
            k_start = base + vb[lvl] * half
            c0, cw = COLS[p]
            out_ref[0, pl.ds(k_start, half), pl.ds(c0, cw)] = (
                out_ref[0, pl.ds(k_start, half), pl.ds(c0, cw)]
                + comm_ref[lvl, pl.ds(0, half), pl.ds(c0, cw)]
            )

        def ag_rdma(lvl, p):
            g = 128 << lvl
            rs_lvl = 2 - lvl
            vb = vbits(p)
            start = sum((vb[j] * (512 >> j) for j in range(rs_lvl + 1)), 0)
            c0, cw = COLS[p]
            return pltpu.make_async_remote_copy(
                src_ref=out_ref.at[0, pl.ds(start, g), pl.ds(c0, cw)],
                dst_ref=out_ref.at[0, pl.ds(start, g), pl.ds(c0, cw)],
                send_sem=ag_send.at[p, lvl],
                recv_sem=ag_recv.at[p, lvl],
                device_id=(my_pos ^ ORDERS[p][rs_lvl],),
                device_id_type=pl.DeviceIdType.MESH,
            )

        cp_wo.wait()
        rs_d = {}
        for p in range(3):
            vb = vbits(p)
            c0, cw = COLS[p]
            s_start = (1 - vb[0]) * HR
            out_ref[0, pl.ds(s_start, HR), pl.ds(c0, cw)] = jnp.dot(
                ctx_ref[pl.ds(s_start, HR), :], wo_ref[:, c0:c0 + cw],
                preferred_element_type=jnp.float32)
            rs_d[(0, p)] = rs_rdma(0, p)
            rs_d[(0, p)].start()
        for p in range(3):
            vb = vbits(p)
            c0, cw = COLS[p]
            k_start = vb[0] * HR
            out_ref[0, pl.ds(k_start, HR), pl.ds(c0, cw)] = jnp.dot(
                ctx_ref[pl.ds(k_start, HR), :], wo_ref[:, c0:c0 + cw],
                preferred_element_type=jnp.float32)

        ag_d = {}
        for lvl in range(3):
            for p in range(3):
                rs_d[(lvl, p)].wait()
                rs_add(lvl, p)
                if lvl < 2:
                    rs_d[(lvl + 1, p)] = rs_rdma(lvl + 1, p)
                    rs_d[(lvl + 1, p)].start()
                else:
                    ag_d[(0, p)] = ag_rdma(0, p)
                    ag_d[(0, p)].start()
        for lvl in range(3):
            for p in range(3):
                ag_d[(lvl, p)].wait()
                if lvl < 2:
                    ag_d[(lvl + 1, p)] = ag_rdma(lvl + 1, p)
                    ag_d[(lvl + 1, p)].start()

    return pl.pallas_call(
        body,
        out_shape=jax.ShapeDtypeStruct((1, SQ, Wo.shape[1]), jnp.float32),
        in_specs=[pl.BlockSpec(memory_space=pltpu.VMEM)] * 2
        + [pl.BlockSpec(memory_space=pl.ANY)] * 3,
        out_specs=pl.BlockSpec(memory_space=pltpu.VMEM),
        scratch_shapes=[
            pltpu.VMEM((3, 512, 1024), jnp.float32),
            pltpu.VMEM((SQ, 1024), jnp.float32),
            pltpu.VMEM((1, SKV, H_LOC, DH), jnp.float32),
            pltpu.VMEM((1, SKV, H_LOC, DH), jnp.float32),
            pltpu.VMEM((D_LOC, 1024), jnp.float32),
            pltpu.SemaphoreType.DMA((3,)),
            pltpu.SemaphoreType.DMA((3, 3)),
            pltpu.SemaphoreType.DMA((3, 3)),
            pltpu.SemaphoreType.DMA((3, 3)),
            pltpu.SemaphoreType.DMA((3, 3)),
        ],
        compiler_params=pltpu.CompilerParams(collective_id=0),
    )(x, Wq_loc, K_ext, V_ext, Wo_loc)


# device time: 67120 ns/iter; 1.2295x vs baseline; 1.0358x over previous
import jax
import jax.numpy as jnp
from jax import lax
from jax.experimental import pallas as pl
from jax.experimental.pallas import tpu as pltpu

N_DEV = 8
SQ = 1024
SKV = 1024
H_LOC = 8
DH = 128
D_LOC = H_LOC * DH
BLK = 64
SCALE = 0.08838834764831843

ORDERS = [[1, 3, 4], [3, 4, 1], [4, 1, 3]]
COLS = [(0, 384), (384, 384), (768, 256)]


def kernel(x, Wq, K_ext, V_ext, Wo):
    def body(x_ref, wq_hbm, k_hbm, v_hbm, wo_hbm, out_ref,
             comm_ref, ctx_ref, wq_ref, k_ref, v_ref, wo_ref, cp_sems,
             rs_send, rs_recv, ag_send, ag_recv):
        my_pos = lax.axis_index("i")

        cp_wq = pltpu.make_async_copy(
            wq_hbm.at[:, pl.ds(my_pos * D_LOC, D_LOC)], wq_ref, cp_sems.at[3])
        cp_wq.start()
        cp_k = pltpu.make_async_copy(k_hbm, k_ref, cp_sems.at[0])
        cp_v = pltpu.make_async_copy(v_hbm, v_ref, cp_sems.at[1])
        cp_wo = pltpu.make_async_copy(
            wo_hbm.at[pl.ds(my_pos * D_LOC, D_LOC), :], wo_ref, cp_sems.at[2])
        cp_k.start()
        cp_v.start()
        cp_wo.start()
        b0 = my_pos % 2
        b1 = (my_pos // 2) % 2
        b2 = my_pos // 4
        dual = {1: b0 ^ b1, 3: b1, 4: b2}

        barrier_sem = pltpu.get_barrier_semaphore()
        for m in (1, 3, 4):
            pl.semaphore_signal(barrier_sem, inc=1,
                                device_id=(my_pos ^ m,),
                                device_id_type=pl.DeviceIdType.MESH)
        pl.semaphore_wait(barrier_sem, 3)

        xm = x_ref[0]
        cp_wq.wait()
        Q = jnp.dot(xm, wq_ref[...], preferred_element_type=jnp.float32) * SCALE

        HR = SQ // 2
        QR = SQ // 4
        masks = []
        for qi in range(4):
            kl = QR * (qi + 1)
            rbq = qi * (QR // BLK) + lax.broadcasted_iota(
                jnp.int32, (QR, kl), 0) // BLK
            cbq = lax.broadcasted_iota(jnp.int32, (QR, kl), 1) // BLK
            masks.append(cbq <= rbq)

        cp_k.wait()
        cp_v.wait()
        for h in range(H_LOC):
            k = k_ref[0, :, h, :]
            v = v_ref[0, :, h, :]
            for qi in range(4):
                kl = QR * (qi + 1)
                q = Q[qi * QR:(qi + 1) * QR, h * DH:(h + 1) * DH]
                s = lax.dot_general(q, k[:kl, :], (((1,), (1,)), ((), ())),
                                    preferred_element_type=jnp.float32)
                w = jnp.where(masks[qi], jnp.exp(s), 0.0)
                denom = jnp.sum(w, axis=-1, keepdims=True)
                ctxh = jnp.dot(w, v[:kl, :], preferred_element_type=jnp.float32)
                ctx_ref[qi * QR:(qi + 1) * QR, h * DH:(h + 1) * DH] = ctxh / denom

        def vbits(p):
            return [dual[ORDERS[p][j]] for j in range(3)]

        def rs_rdma(lvl, p):
            half = 512 >> lvl
            vb = vbits(p)
            base = sum((vb[j] * (512 >> j) for j in range(lvl)), 0)
            s_start = base + (1 - vb[lvl]) * half
            c0, cw = COLS[p]
            return pltpu.make_async_remote_copy(
                src_ref=out_ref.at[0, pl.ds(s_start, half), pl.ds(c0, cw)],
                dst_ref=comm_ref.at[lvl, pl.ds(0, half), pl.ds(c0, cw)],
                send_sem=rs_send.at[p, lvl],
                recv_sem=rs_recv.at[p, lvl],
                device_id=(my_pos ^ ORDERS[p][lvl],),
                device_id_type=pl.DeviceIdType.MESH,
            )

        def rs_add(lvl, p):
            half = 512 >> lvl
            vb = vbits(p)
            base = sum((vb[j] * (512 >> j) for j in range(lvl)), 0)
            k_start = base + vb[lvl] * half
            c0, cw = COLS[p]
            out_ref[0, pl.ds(k_start, half), pl.ds(c0, cw)] = (
                out_ref[0, pl.ds(k_start, half), pl.ds(c0, cw)]
                + comm_ref[lvl, pl.ds(0, half), pl.ds(c0, cw)]
            )

        def ag_rdma(lvl, p):
            g = 128 << lvl
            rs_lvl = 2 - lvl
            vb = vbits(p)
            start = sum((vb[j] * (512 >> j) for j in range(rs_lvl + 1)), 0)
            c0, cw = COLS[p]
            return pltpu.make_async_remote_copy(
                src_ref=out_ref.at[0, pl.ds(start, g), pl.ds(c0, cw)],
                dst_ref=out_ref.at[0, pl.ds(start, g), pl.ds(c0, cw)],
                send_sem=ag_send.at[p, lvl],
                recv_sem=ag_recv.at[p, lvl],
                device_id=(my_pos ^ ORDERS[p][rs_lvl],),
                device_id_type=pl.DeviceIdType.MESH,
            )

        cp_wo.wait()
        rs_d = {}
        for p in range(3):
            vb = vbits(p)
            c0, cw = COLS[p]
            s_start = (1 - vb[0]) * HR
            out_ref[0, pl.ds(s_start, HR), pl.ds(c0, cw)] = jnp.dot(
                ctx_ref[pl.ds(s_start, HR), :], wo_ref[:, c0:c0 + cw],
                preferred_element_type=jnp.float32)
            rs_d[(0, p)] = rs_rdma(0, p)
            rs_d[(0, p)].start()
        for p in range(3):
            vb = vbits(p)
            c0, cw = COLS[p]
            k_start = vb[0] * HR
            out_ref[0, pl.ds(k_start, HR), pl.ds(c0, cw)] = jnp.dot(
                ctx_ref[pl.ds(k_start, HR), :], wo_ref[:, c0:c0 + cw],
                preferred_element_type=jnp.float32)

        ag_d = {}
        for lvl in range(3):
            for p in range(3):
                rs_d[(lvl, p)].wait()
                rs_add(lvl, p)
                if lvl < 2:
                    rs_d[(lvl + 1, p)] = rs_rdma(lvl + 1, p)
                    rs_d[(lvl + 1, p)].start()
                else:
                    ag_d[(0, p)] = ag_rdma(0, p)
                    ag_d[(0, p)].start()
        for lvl in range(3):
            for p in range(3):
                ag_d[(lvl, p)].wait()
                if lvl < 2:
                    ag_d[(lvl + 1, p)] = ag_rdma(lvl + 1, p)
                    ag_d[(lvl + 1, p)].start()

    return pl.pallas_call(
        body,
        out_shape=jax.ShapeDtypeStruct((1, SQ, Wo.shape[1]), jnp.float32),
        in_specs=[pl.BlockSpec(memory_space=pltpu.VMEM)]
        + [pl.BlockSpec(memory_space=pl.ANY)] * 4,
        out_specs=pl.BlockSpec(memory_space=pltpu.VMEM),
        scratch_shapes=[
            pltpu.VMEM((3, 512, 1024), jnp.float32),
            pltpu.VMEM((SQ, 1024), jnp.float32),
            pltpu.VMEM((1024, D_LOC), jnp.float32),
            pltpu.VMEM((1, SKV, H_LOC, DH), jnp.float32),
            pltpu.VMEM((1, SKV, H_LOC, DH), jnp.float32),
            pltpu.VMEM((D_LOC, 1024), jnp.float32),
            pltpu.SemaphoreType.DMA((4,)),
            pltpu.SemaphoreType.DMA((3, 3)),
            pltpu.SemaphoreType.DMA((3, 3)),
            pltpu.SemaphoreType.DMA((3, 3)),
            pltpu.SemaphoreType.DMA((3, 3)),
        ],
        compiler_params=pltpu.CompilerParams(collective_id=0),
    )(x, Wq, K_ext, V_ext, Wo)
